# Initial kernel scaffold; baseline (speedup 1.0000x reference)
import jax
import jax.numpy as jnp
from jax.experimental import pallas as pl


def kernel(x, edge_index, W1, b1, W2, b2):
    raise NotImplementedError("write your pallas kernel here")


# speedup vs baseline: 55.4633x; 55.4633x over previous
import functools

import jax
import jax.numpy as jnp
from jax import lax
from jax.experimental import pallas as pl
from jax.experimental.pallas import tpu as pltpu
from jax.experimental.pallas import tpu_sc as plsc

NC = 2
NS = 16
NW = NC * NS
LANE = 128
CHUNK = 8
F = 4

N_IN = 100000
NPAD = 102400
RPW = 784
EPAD = NW * RPW * LANE
SLICE = NPAD // NS
BLK = 6400


def _sc_mesh():
    return plsc.VectorSubcoreMesh(core_axis_name="c", subcore_axis_name="s")


_SC_PARAMS = pltpu.CompilerParams(use_tc_tiling_on_sc=False)


def _worker_ids():
    cid = lax.axis_index("c")
    sid = lax.axis_index("s")
    return cid, sid, sid * NC + cid


def _zero_acc(zeros_h, acc, sid):
    sl = pl.ds(sid * SLICE, SLICE)
    pltpu.sync_copy(zeros_h.at[sl], acc.at[sl])


def _copy_out(acc, out_h, cid, sid):
    sl = pl.ds(sid * SLICE, SLICE)
    pltpu.sync_copy(acc.at[sl], out_h.at[cid].at[sl])


def _deg_kernel(ones_h, dst_h, zeros_h, out_h, acc, idx_d, ones_v, sem_i, sem_s):
    cid, sid, wid = _worker_ids()
    _zero_acc(zeros_h, acc, sid)
    pltpu.sync_copy(ones_h, ones_v)
    plsc.subcore_barrier()

    @pl.loop(0, RPW, step=CHUNK)
    def _(i):
        pltpu.async_copy(dst_h.at[wid].at[pl.ds(i, CHUNK)], idx_d, sem_i).wait()
        adds = [
            pltpu.async_copy(ones_v, acc.at[idx_d.at[j]], sem_s, add=True)
            for j in range(CHUNK)
        ]
        for a in adds:
            a.wait()

    plsc.subcore_barrier()
    _copy_out(acc, out_h, cid, sid)


def _agg_kernel(table_h, src_h, dst_h, zeros_h, out_h,
                acc, idx_s, idx_d, rows, sem_i, sem_g, sem_s):
    cid, sid, wid = _worker_ids()
    _zero_acc(zeros_h, acc, sid)
    plsc.subcore_barrier()

    @pl.loop(0, RPW, step=CHUNK)
    def _(i):
        ci = pltpu.async_copy(src_h.at[wid].at[pl.ds(i, CHUNK)], idx_s, sem_i)
        cj = pltpu.async_copy(dst_h.at[wid].at[pl.ds(i, CHUNK)], idx_d, sem_i)
        ci.wait()
        cj.wait()
        gets = [
            pltpu.async_copy(table_h.at[idx_s.at[j]], rows.at[j], sem_g)
            for j in range(CHUNK)
        ]
        for g in gets:
            g.wait()
        adds = [
            pltpu.async_copy(rows.at[j], acc.at[idx_d.at[j]], sem_s, add=True)
            for j in range(CHUNK)
        ]
        for a in adds:
            a.wait()

    plsc.subcore_barrier()
    _copy_out(acc, out_h, cid, sid)


def _sc_degree(dst3, zeros, ones):
    kern = pl.kernel(
        _deg_kernel,
        out_type=jax.ShapeDtypeStruct((NC, NPAD, F), jnp.float32),
        mesh=_sc_mesh(),
        scratch_types=[
            pltpu.VMEM_SHARED((NPAD, F), jnp.float32),
            pltpu.VMEM((CHUNK, LANE), jnp.int32),
            pltpu.VMEM((LANE, F), jnp.float32),
            pltpu.SemaphoreType.DMA,
            pltpu.SemaphoreType.DMA,
        ],
        compiler_params=_SC_PARAMS,
    )
    return kern(ones, dst3, zeros)


def _sc_aggregate(table, src3, dst3, zeros):
    kern = pl.kernel(
        _agg_kernel,
        out_type=jax.ShapeDtypeStruct((NC, NPAD, F), jnp.float32),
        mesh=_sc_mesh(),
        scratch_types=[
            pltpu.VMEM_SHARED((NPAD, F), jnp.float32),
            pltpu.VMEM((CHUNK, LANE), jnp.int32),
            pltpu.VMEM((CHUNK, LANE), jnp.int32),
            pltpu.VMEM((CHUNK, LANE, F), jnp.float32),
            pltpu.SemaphoreType.DMA,
            pltpu.SemaphoreType.DMA,
            pltpu.SemaphoreType.DMA,
        ],
        compiler_params=_SC_PARAMS,
    )
    return kern(table, src3, dst3, zeros)


def _tc_prescale_body(deg0, deg1, x, d_out, y1_out):
    deg = deg0[:, 0:1] + deg1[:, 0:1] + 1.0
    d = lax.rsqrt(deg)
    d_out[...] = d
    y1_out[...] = x[...] * d


def _tc_prescale(deg_parts, x_pad):
    grid = NPAD // BLK
    spec4 = pl.BlockSpec((BLK, F), lambda i: (i, 0))
    spec1 = pl.BlockSpec((BLK, 1), lambda i: (i, 0))
    return pl.pallas_call(
        _tc_prescale_body,
        grid=(grid,),
        in_specs=[spec4, spec4, spec4],
        out_specs=[spec1, spec4],
        out_shape=[
            jax.ShapeDtypeStruct((NPAD, 1), jnp.float32),
            jax.ShapeDtypeStruct((NPAD, F), jnp.float32),
        ],
    )(deg_parts[0], deg_parts[1], x_pad)


def _tc_mid_body(a0, a1, x, d, w1, b1, w2, y2_out, self2_out):
    dv = d[...]
    pre = (a0[...] + a1[...]) * dv + x[...] * (dv * dv)
    h1 = jnp.maximum(jnp.dot(pre, w1[...], preferred_element_type=jnp.float32)
                     + b1[...], 0.0)
    s = jnp.dot(h1, w2[...], preferred_element_type=jnp.float32)
    y2_out[...] = jnp.broadcast_to(s * dv, (BLK, F))
    self2_out[...] = s * dv * dv


def _tc_mid(a1_parts, x_pad, d, W1, b1, W2):
    grid = NPAD // BLK
    spec4 = pl.BlockSpec((BLK, F), lambda i: (i, 0))
    spec1 = pl.BlockSpec((BLK, 1), lambda i: (i, 0))
    w1s = pl.BlockSpec((F, 64), lambda i: (0, 0))
    b1s = pl.BlockSpec((1, 64), lambda i: (0, 0))
    w2s = pl.BlockSpec((64, 1), lambda i: (0, 0))
    return pl.pallas_call(
        _tc_mid_body,
        grid=(grid,),
        in_specs=[spec4, spec4, spec4, spec1, w1s, b1s, w2s],
        out_specs=[spec4, spec1],
        out_shape=[
            jax.ShapeDtypeStruct((NPAD, F), jnp.float32),
            jax.ShapeDtypeStruct((NPAD, 1), jnp.float32),
        ],
    )(a1_parts[0], a1_parts[1], x_pad, d, W1, b1.reshape(1, 64), W2)


def _tc_final_body(a0, a1, d, self2, b2, out):
    agg = (a0[:, 0:1] + a1[:, 0:1]) * d[...]
    out[...] = jax.nn.sigmoid(agg + self2[...] + b2[...])


def _tc_final(a2_parts, d, self2, b2):
    grid = NPAD // BLK
    spec4 = pl.BlockSpec((BLK, F), lambda i: (i, 0))
    spec1 = pl.BlockSpec((BLK, 1), lambda i: (i, 0))
    b2s = pl.BlockSpec((1, 1), lambda i: (0, 0))
    return pl.pallas_call(
        _tc_final_body,
        grid=(grid,),
        in_specs=[spec4, spec4, spec1, spec1, b2s],
        out_specs=spec1,
        out_shape=jax.ShapeDtypeStruct((NPAD, 1), jnp.float32),
    )(a2_parts[0], a2_parts[1], d, self2, b2.reshape(1, 1))


@jax.jit
def kernel(x, edge_index, W1, b1, W2, b2):
    n = x.shape[0]
    e = edge_index.shape[1]

    e32 = edge_index.astype(jnp.int32)
    fill = jnp.full((EPAD - e,), n, dtype=jnp.int32)
    src3 = jnp.concatenate([e32[0], fill]).reshape(NW, RPW, LANE)
    dst3 = jnp.concatenate([e32[1], fill]).reshape(NW, RPW, LANE)

    x_pad = jnp.zeros((NPAD, F), jnp.float32).at[:n].set(x)
    zeros = jnp.zeros((NPAD, F), jnp.float32)
    ones = jnp.ones((LANE, F), jnp.float32)

    deg_parts = _sc_degree(dst3, zeros, ones)
    d, y1 = _tc_prescale(deg_parts, x_pad)
    a1_parts = _sc_aggregate(y1, src3, dst3, zeros)
    y2, self2 = _tc_mid(a1_parts, x_pad, d, W1, b1, W2)
    a2_parts = _sc_aggregate(y2, src3, dst3, zeros)
    out = _tc_final(a2_parts, d, self2, b2)
    return out[:n]
